# baseline (speedup 1.0000x reference)
import jax
import jax.numpy as jnp
from jax import lax
from jax.experimental import pallas as pl
from jax.experimental.pallas import tpu as pltpu
from jax.experimental.pallas import tpu_sc as plsc

_EPS = 1e-5
_NC = 2
_NT = 16
_CK = 128


def _sc_segsum(table, src2, dst3, n_chunks, acc_rows):
  h = table.shape[1]
  rt = acc_rows // _NT
  mesh = plsc.VectorSubcoreMesh(
      core_axis_name="c", subcore_axis_name="s", num_cores=_NC,
      num_subcores=_NT)
  zac = jnp.zeros((rt, h), jnp.float32)

  nw = n_chunks // 8

  def body(tab, src_h, dst_h, zac_h, agg_o, src_v, dst_v, gb0, gb1, acc,
           gsem0, gsem1, ssem0, ssem1, isem0, isem1):
    c = lax.axis_index("c")
    s = lax.axis_index("s")
    base = s * rt
    gb = (gb0, gb1)
    gsem = (gsem0, gsem1)
    ssem = (ssem0, ssem1)
    isem = (isem0, isem1)

    def idx_start(w, b):
      pltpu.async_copy(src_h.at[c, s, pl.ds(w * 8, 8)], src_v.at[b], isem[b])
      pltpu.async_copy(dst_h.at[s, pl.ds(w * 8, 8)], dst_v.at[b], isem[b])

    def idx_wait(w, b):
      pltpu.make_async_copy(src_h.at[c, s, pl.ds(w * 8, 8)], src_v.at[b],
                            isem[b]).wait()
      pltpu.make_async_copy(dst_h.at[s, pl.ds(w * 8, 8)], dst_v.at[b],
                            isem[b]).wait()

    idx_start(0, 0)
    idx_start(1, 1)
    pltpu.sync_copy(zac_h, acc.at[pl.ds(base, rt)])
    plsc.subcore_barrier()

    def window_pair(j2, carry):
      for b in range(2):
        w = 2 * j2 + b
        idx_wait(w, b)
        g_cp = [None] * 8
        s_cp = [None] * 8
        g_cp[0] = pltpu.async_copy(tab.at[src_v.at[b, 0]], gb[0], gsem[0])
        for k in range(8):
          g_cp[k].wait()
          if k < 7:
            if k >= 1:
              s_cp[k - 1].wait()
            g_cp[k + 1] = pltpu.async_copy(tab.at[src_v.at[b, k + 1]],
                                           gb[(k + 1) % 2], gsem[(k + 1) % 2])
          s_cp[k] = pltpu.async_copy(gb[k % 2], acc.at[dst_v.at[b, k]],
                                     ssem[k % 2], add=True)
        s_cp[6].wait()
        s_cp[7].wait()

        @pl.when(w + 2 < nw)
        def _():
          idx_start(w + 2, b)
      return carry

    lax.fori_loop(0, nw // 2, window_pair, 0)
    plsc.subcore_barrier()
    pltpu.sync_copy(acc.at[pl.ds(base, rt)], agg_o.at[c, pl.ds(base, rt)])

  return pl.kernel(
      body,
      out_type=[jax.ShapeDtypeStruct((_NC, acc_rows, h), jnp.float32)],
      mesh=mesh,
      scratch_types=[
          pltpu.VMEM((2, 8, _CK), jnp.int32),
          pltpu.VMEM((2, 8, _CK), jnp.int32),
          pltpu.VMEM((_CK, h), jnp.float32),
          pltpu.VMEM((_CK, h), jnp.float32),
          pltpu.VMEM_SHARED((acc_rows, h), jnp.float32),
          pltpu.SemaphoreType.DMA,
          pltpu.SemaphoreType.DMA,
          pltpu.SemaphoreType.DMA,
          pltpu.SemaphoreType.DMA,
          pltpu.SemaphoreType.DMA,
          pltpu.SemaphoreType.DMA,
      ])(table, src2, dst3, zac)[0]


def _sc_degree(dst3, n_chunks, acc_rows):
  h = 128
  rt = acc_rows // _NT
  half = n_chunks // 2
  mesh = plsc.VectorSubcoreMesh(
      core_axis_name="c", subcore_axis_name="s", num_cores=_NC,
      num_subcores=_NT)
  zac = jnp.zeros((rt, h), jnp.float32)
  ones = jnp.ones((_CK, h), jnp.float32)

  def body(dst_h, zac_h, ones_h, deg_o, dst_v, ones_v, dacc):
    c = lax.axis_index("c")
    s = lax.axis_index("s")
    base = s * rt
    pltpu.sync_copy(zac_h, dacc.at[pl.ds(base, rt)])
    pltpu.sync_copy(ones_h, ones_v)
    plsc.subcore_barrier()

    def window(j, carry):
      pltpu.sync_copy(dst_h.at[s, pl.ds(c * half + j * 8, 8)], dst_v)
      for k in range(8):
        pltpu.sync_copy(ones_v, dacc.at[dst_v.at[k]], add=True)
      return carry

    lax.fori_loop(0, half // 8, window, 0)
    plsc.subcore_barrier()
    pltpu.sync_copy(dacc.at[pl.ds(base, rt)], deg_o.at[c, pl.ds(base, rt)])

  return pl.kernel(
      body,
      out_type=[jax.ShapeDtypeStruct((_NC, acc_rows, h), jnp.float32)],
      mesh=mesh,
      scratch_types=[
          pltpu.VMEM((8, _CK), jnp.int32),
          pltpu.VMEM((_CK, h), jnp.float32),
          pltpu.VMEM_SHARED((acc_rows, h), jnp.float32),
      ])(dst3, zac, ones)[0]


def _prep_call(x, y, nb, bn):
  n, d = x.shape
  h = d // 2

  def body(x_ref, y_ref, o_ref):
    p = x_ref[...] * y_ref[...]
    o_ref[0] = p[:, :h]
    o_ref[1] = p[:, h:]

  return pl.pallas_call(
      body,
      grid=(nb,),
      in_specs=[pl.BlockSpec((bn, d), lambda i: (i, 0)),
                pl.BlockSpec((bn, d), lambda i: (i, 0))],
      out_specs=pl.BlockSpec((2, bn, h), lambda i: (0, i, 0)),
      out_shape=jax.ShapeDtypeStruct((2, n, h), jnp.float32),
  )(x, y)


def _mm_call(tabs, deg, w, b, specs, n, nb, bn):
  n_tabs = len(tabs)
  n_ops = len(specs)
  d = w.shape[-1]
  h = d // 2
  acc_rows = tabs[0].shape[1]

  def body(*refs):
    tab_refs = refs[:n_tabs]
    d_ref = refs[n_tabs]
    w_ref = refs[n_tabs + 1]
    b_ref = refs[n_tabs + 2]
    y_refs = refs[n_tabs + 3:n_tabs + 3 + n_ops]
    st_ref = refs[n_tabs + 3 + n_ops]
    acc_ref = refs[-1]
    i = pl.program_id(0)
    rs = 1.0 / jnp.clip(d_ref[0, :, 0:1] + d_ref[1, :, 0:1], 1.0, None)
    parts = []
    for k, (ti, wi) in enumerate(specs):
      a = tab_refs[ti]
      y = (jnp.dot(a[0], w_ref[wi, :h, :], preferred_element_type=jnp.float32)
           + jnp.dot(a[1], w_ref[wi, h:, :],
                     preferred_element_type=jnp.float32))
      y = y * rs + b_ref[wi:wi + 1]
      y_refs[k][...] = y
      parts.append(jnp.sum(y, axis=0, keepdims=True))
      parts.append(jnp.sum(y * y, axis=0, keepdims=True))
    ps = jnp.concatenate(parts, axis=0)

    @pl.when(i == 0)
    def _():
      acc_ref[...] = ps

    @pl.when(i > 0)
    def _():
      acc_ref[...] = acc_ref[...] + ps

    @pl.when(i == nb - 1)
    def _():
      st_ref[...] = acc_ref[...]

  in_specs = (
      [pl.BlockSpec((2, bn, h), lambda i: (0, i, 0)) for _ in range(n_tabs)]
      + [pl.BlockSpec((2, bn, 128), lambda i: (0, i, 0)),
         pl.BlockSpec(w.shape, lambda i: (0, 0, 0)),
         pl.BlockSpec(b.shape, lambda i: (0, 0))])
  out_specs = (
      [pl.BlockSpec((bn, d), lambda i: (i, 0)) for _ in range(n_ops)]
      + [pl.BlockSpec((2 * n_ops, d), lambda i: (0, 0))])
  out_shape = ([jax.ShapeDtypeStruct((n, d), jnp.float32)
                for _ in range(n_ops)]
               + [jax.ShapeDtypeStruct((2 * n_ops, d), jnp.float32)])
  res = pl.pallas_call(
      body,
      grid=(nb,),
      in_specs=in_specs,
      out_specs=out_specs,
      out_shape=out_shape,
      scratch_shapes=[pltpu.VMEM((2 * n_ops, d), jnp.float32)],
  )(*tabs, deg, w, b)
  return res[:n_ops], res[n_ops]


def _bn(y, st_ref, row, g_ref, be_ref, grow, n):
  mu = st_ref[2 * row:2 * row + 1] / n
  var = st_ref[2 * row + 1:2 * row + 2] / n - mu * mu
  inv = g_ref[grow:grow + 1] * lax.rsqrt(var + _EPS)
  return jnp.maximum((y - mu) * inv + be_ref[grow:grow + 1], 0.0)


def _stage1_call(y0, st0, g, be, x, n, nb, bn):
  d = y0.shape[1]
  h = d // 2

  def body(y_ref, st_ref, g_ref, be_ref, x_ref, z_ref, p1_ref, p2_ref):
    z = _bn(y_ref[...], st_ref, 0, g_ref, be_ref, 0, n)
    z_ref[...] = z
    p1 = x_ref[...] * z
    p2 = z * z
    p1_ref[0] = p1[:, :h]
    p1_ref[1] = p1[:, h:]
    p2_ref[0] = p2[:, :h]
    p2_ref[1] = p2[:, h:]

  return pl.pallas_call(
      body,
      grid=(nb,),
      in_specs=[pl.BlockSpec((bn, d), lambda i: (i, 0)),
                pl.BlockSpec(st0.shape, lambda i: (0, 0)),
                pl.BlockSpec(g.shape, lambda i: (0, 0)),
                pl.BlockSpec(be.shape, lambda i: (0, 0)),
                pl.BlockSpec((bn, d), lambda i: (i, 0))],
      out_specs=[pl.BlockSpec((bn, d), lambda i: (i, 0)),
                 pl.BlockSpec((2, bn, h), lambda i: (0, i, 0)),
                 pl.BlockSpec((2, bn, h), lambda i: (0, i, 0))],
      out_shape=[jax.ShapeDtypeStruct((n, d), jnp.float32),
                 jax.ShapeDtypeStruct((2, n, h), jnp.float32),
                 jax.ShapeDtypeStruct((2, n, h), jnp.float32)],
  )(y0, st0, g, be, x)


def _stage2_call(y10, y11, y20, st, g, be, z, n, nb, bn):
  d = y10.shape[1]
  h = d // 2

  def body(y10_r, y11_r, y20_r, st_ref, g_ref, be_ref, z_ref,
           s2_ref, h20_ref, p3_ref):
    h10 = _bn(y10_r[...], st_ref, 0, g_ref, be_ref, 1, n)
    h11 = _bn(y11_r[...], st_ref, 1, g_ref, be_ref, 2, n)
    h20 = _bn(y20_r[...], st_ref, 2, g_ref, be_ref, 3, n)
    s2 = h10 + h11
    s2_ref[...] = s2
    h20_ref[...] = h20
    p3 = s2 * z_ref[...]
    p3_ref[0] = p3[:, :h]
    p3_ref[1] = p3[:, h:]

  return pl.pallas_call(
      body,
      grid=(nb,),
      in_specs=[pl.BlockSpec((bn, d), lambda i: (i, 0)) for _ in range(3)]
      + [pl.BlockSpec(st.shape, lambda i: (0, 0)),
         pl.BlockSpec(g.shape, lambda i: (0, 0)),
         pl.BlockSpec(be.shape, lambda i: (0, 0)),
         pl.BlockSpec((bn, d), lambda i: (i, 0))],
      out_specs=[pl.BlockSpec((bn, d), lambda i: (i, 0)),
                 pl.BlockSpec((bn, d), lambda i: (i, 0)),
                 pl.BlockSpec((2, bn, h), lambda i: (0, i, 0))],
      out_shape=[jax.ShapeDtypeStruct((n, d), jnp.float32),
                 jax.ShapeDtypeStruct((n, d), jnp.float32),
                 jax.ShapeDtypeStruct((2, n, h), jnp.float32)],
  )(y10, y11, y20, st, g, be, z)


def _stage3_call(y22, st22, g, be, h20, z, s2, wc, bc, n, nb, bn):
  d = y22.shape[1]

  def body(y22_r, st_ref, g_ref, be_ref, h20_r, z_r, s2_r, wc_ref, bc_ref,
           yc_ref, stc_ref, acc_ref):
    i = pl.program_id(0)
    h22 = _bn(y22_r[...], st_ref, 0, g_ref, be_ref, 4, n)
    s3 = h20_r[...] + h22
    yc = (jnp.dot(z_r[...], wc_ref[:d, :], preferred_element_type=jnp.float32)
          + jnp.dot(s2_r[...], wc_ref[d:2 * d, :],
                    preferred_element_type=jnp.float32)
          + jnp.dot(s3, wc_ref[2 * d:, :], preferred_element_type=jnp.float32)
          + bc_ref[0:1])
    yc_ref[...] = yc
    ps = jnp.concatenate([jnp.sum(yc, axis=0, keepdims=True),
                          jnp.sum(yc * yc, axis=0, keepdims=True)], axis=0)

    @pl.when(i == 0)
    def _():
      acc_ref[...] = ps

    @pl.when(i > 0)
    def _():
      acc_ref[...] = acc_ref[...] + ps

    @pl.when(i == nb - 1)
    def _():
      stc_ref[...] = acc_ref[...]

  return pl.pallas_call(
      body,
      grid=(nb,),
      in_specs=[pl.BlockSpec((bn, d), lambda i: (i, 0)),
                pl.BlockSpec(st22.shape, lambda i: (0, 0)),
                pl.BlockSpec(g.shape, lambda i: (0, 0)),
                pl.BlockSpec(be.shape, lambda i: (0, 0)),
                pl.BlockSpec((bn, d), lambda i: (i, 0)),
                pl.BlockSpec((bn, d), lambda i: (i, 0)),
                pl.BlockSpec((bn, d), lambda i: (i, 0)),
                pl.BlockSpec(wc.shape, lambda i: (0, 0)),
                pl.BlockSpec(bc.shape, lambda i: (0, 0))],
      out_specs=[pl.BlockSpec((bn, d), lambda i: (i, 0)),
                 pl.BlockSpec((2, d), lambda i: (0, 0))],
      out_shape=[jax.ShapeDtypeStruct((n, d), jnp.float32),
                 jax.ShapeDtypeStruct((2, d), jnp.float32)],
      scratch_shapes=[pltpu.VMEM((2, d), jnp.float32)],
  )(y22, st22, g, be, h20, z, s2, wc, bc)


def _final_call(yc, stc, gc, bc, n, nb, bn):
  d = yc.shape[1]

  def body(y_ref, st_ref, g_ref, be_ref, o_ref):
    o_ref[...] = _bn(y_ref[...], st_ref, 0, g_ref, be_ref, 0, n)

  return pl.pallas_call(
      body,
      grid=(nb,),
      in_specs=[pl.BlockSpec((bn, d), lambda i: (i, 0)),
                pl.BlockSpec(stc.shape, lambda i: (0, 0)),
                pl.BlockSpec(gc.shape, lambda i: (0, 0)),
                pl.BlockSpec(bc.shape, lambda i: (0, 0))],
      out_specs=pl.BlockSpec((bn, d), lambda i: (i, 0)),
      out_shape=jax.ShapeDtypeStruct((n, d), jnp.float32),
  )(yc, stc, gc, bc)


def kernel(src_emb, hr, edge_index, W_ops, b_ops, gamma_ops, beta_ops,
           W_cat, b_cat, gamma_c, beta_c):
  n, d = src_emb.shape
  e = edge_index.shape[1]
  h = d // 2

  bn = 1000
  nb = n // bn

  per_tile = -(-e // (_NT * _CK * 16)) * _CK * 16
  n_chunks = per_tile // _CK
  ep = per_tile * _NT
  pad = ep - e
  src = edge_index[0]
  dst = edge_index[1]
  srcp = jnp.concatenate([src, jnp.zeros((pad,), jnp.int32)])
  dstp = jnp.concatenate([dst, jnp.full((pad,), n, jnp.int32)])
  src2 = jnp.stack([srcp, srcp + n]).reshape(_NC, _NT, n_chunks, _CK)
  dst3 = dstp.reshape(_NT, n_chunks, _CK)
  rt = -(-(n + 1) // (_NT * 8)) * 8
  acc_rows = rt * _NT

  b2 = b_ops.reshape(5, d)
  g2 = gamma_ops.reshape(5, d)
  be2 = beta_ops.reshape(5, d)
  bc2 = b_cat.reshape(1, d)
  gc2 = gamma_c.reshape(1, d)
  bec2 = beta_c.reshape(1, d)

  p0 = _prep_call(src_emb, hr, nb, bn)
  deg = _sc_degree(dst3, n_chunks, acc_rows)
  agg0 = _sc_segsum(p0.reshape(2 * n, h), src2, dst3, n_chunks, acc_rows)

  (y0,), st0 = _mm_call([agg0], deg, W_ops, b2, [(0, 0)], n, nb, bn)
  z, p1, p2 = _stage1_call(y0, st0, g2, be2, src_emb, n, nb, bn)

  agg1 = _sc_segsum(p1.reshape(2 * n, h), src2, dst3, n_chunks, acc_rows)
  agg2 = _sc_segsum(p2.reshape(2 * n, h), src2, dst3, n_chunks, acc_rows)

  (y10, y11, y20), st3 = _mm_call([agg1, agg2], deg, W_ops, b2,
                                  [(0, 1), (1, 2), (0, 3)], n, nb, bn)
  s2, h20, p3 = _stage2_call(y10, y11, y20, st3, g2, be2, z, n, nb, bn)

  agg3 = _sc_segsum(p3.reshape(2 * n, h), src2, dst3, n_chunks, acc_rows)

  (y22,), st22 = _mm_call([agg3], deg, W_ops, b2, [(0, 4)], n, nb, bn)
  yc, stc = _stage3_call(y22, st22, g2, be2, h20, z, s2, W_cat, bc2,
                         n, nb, bn)
  return _final_call(yc, stc, gc2, bec2, n, nb, bn)


# speedup vs baseline: 1.0090x; 1.0090x over previous
import jax
import jax.numpy as jnp
from jax import lax
from jax.experimental import pallas as pl
from jax.experimental.pallas import tpu as pltpu
from jax.experimental.pallas import tpu_sc as plsc

_EPS = 1e-5
_NC = 2
_NT = 16
_CK = 128


def _sc_segsum(table, src2, dst3, n_chunks, acc_rows):
  h = table.shape[1]
  rt = acc_rows // _NT
  mesh = plsc.VectorSubcoreMesh(
      core_axis_name="c", subcore_axis_name="s", num_cores=_NC,
      num_subcores=_NT)
  zac = jnp.zeros((rt, h), jnp.float32)

  nw = n_chunks // 8

  def body(tab, src_h, dst_h, zac_h, agg_o, src_v, dst_v, gb0, gb1, acc,
           gsem0, gsem1, ssem0, ssem1, isem0, isem1):
    c = lax.axis_index("c")
    s = lax.axis_index("s")
    base = s * rt
    gb = (gb0, gb1)
    gsem = (gsem0, gsem1)
    ssem = (ssem0, ssem1)
    isem = (isem0, isem1)

    def idx_start(w, b):
      pltpu.async_copy(src_h.at[c, s, pl.ds(w * 8, 8)], src_v.at[b], isem[b])
      pltpu.async_copy(dst_h.at[s, pl.ds(w * 8, 8)], dst_v.at[b], isem[b])

    def idx_wait(w, b):
      pltpu.make_async_copy(src_h.at[c, s, pl.ds(w * 8, 8)], src_v.at[b],
                            isem[b]).wait()
      pltpu.make_async_copy(dst_h.at[s, pl.ds(w * 8, 8)], dst_v.at[b],
                            isem[b]).wait()

    idx_start(0, 0)
    idx_start(1, 1)
    pltpu.sync_copy(zac_h, acc.at[pl.ds(base, rt)])
    plsc.subcore_barrier()

    def window_pair(j2, carry):
      w0 = 2 * j2
      idx_wait(w0, 0)
      g_cp = [None] * 16
      s_cp = [None] * 16
      g_cp[0] = pltpu.async_copy(tab.at[src_v.at[0, 0]], gb[0], gsem[0])
      for k in range(16):
        g_cp[k].wait()
        if k < 15:
          if k >= 1:
            s_cp[k - 1].wait()
          if k == 7:
            idx_wait(w0 + 1, 1)
          g_cp[k + 1] = pltpu.async_copy(
              tab.at[src_v.at[(k + 1) // 8, (k + 1) % 8]],
              gb[(k + 1) % 2], gsem[(k + 1) % 2])
          if k == 8:
            @pl.when(w0 + 2 < nw)
            def _():
              idx_start(w0 + 2, 0)
        s_cp[k] = pltpu.async_copy(gb[k % 2], acc.at[dst_v.at[k // 8, k % 8]],
                                   ssem[k % 2], add=True)
      s_cp[14].wait()
      s_cp[15].wait()

      @pl.when(w0 + 3 < nw)
      def _():
        idx_start(w0 + 3, 1)
      return carry

    lax.fori_loop(0, nw // 2, window_pair, 0)
    plsc.subcore_barrier()
    pltpu.sync_copy(acc.at[pl.ds(base, rt)], agg_o.at[c, pl.ds(base, rt)])

  return pl.kernel(
      body,
      out_type=[jax.ShapeDtypeStruct((_NC, acc_rows, h), jnp.float32)],
      mesh=mesh,
      scratch_types=[
          pltpu.VMEM((2, 8, _CK), jnp.int32),
          pltpu.VMEM((2, 8, _CK), jnp.int32),
          pltpu.VMEM((_CK, h), jnp.float32),
          pltpu.VMEM((_CK, h), jnp.float32),
          pltpu.VMEM_SHARED((acc_rows, h), jnp.float32),
          pltpu.SemaphoreType.DMA,
          pltpu.SemaphoreType.DMA,
          pltpu.SemaphoreType.DMA,
          pltpu.SemaphoreType.DMA,
          pltpu.SemaphoreType.DMA,
          pltpu.SemaphoreType.DMA,
      ])(table, src2, dst3, zac)[0]


def _sc_degree(dst3, n_chunks, acc_rows):
  h = 128
  rt = acc_rows // _NT
  half = n_chunks // 2
  mesh = plsc.VectorSubcoreMesh(
      core_axis_name="c", subcore_axis_name="s", num_cores=_NC,
      num_subcores=_NT)
  zac = jnp.zeros((rt, h), jnp.float32)
  ones = jnp.ones((_CK, h), jnp.float32)

  def body(dst_h, zac_h, ones_h, deg_o, dst_v, ones_v, dacc):
    c = lax.axis_index("c")
    s = lax.axis_index("s")
    base = s * rt
    pltpu.sync_copy(zac_h, dacc.at[pl.ds(base, rt)])
    pltpu.sync_copy(ones_h, ones_v)
    plsc.subcore_barrier()

    def window(j, carry):
      pltpu.sync_copy(dst_h.at[s, pl.ds(c * half + j * 8, 8)], dst_v)
      for k in range(8):
        pltpu.sync_copy(ones_v, dacc.at[dst_v.at[k]], add=True)
      return carry

    lax.fori_loop(0, half // 8, window, 0)
    plsc.subcore_barrier()
    pltpu.sync_copy(dacc.at[pl.ds(base, rt)], deg_o.at[c, pl.ds(base, rt)])

  return pl.kernel(
      body,
      out_type=[jax.ShapeDtypeStruct((_NC, acc_rows, h), jnp.float32)],
      mesh=mesh,
      scratch_types=[
          pltpu.VMEM((8, _CK), jnp.int32),
          pltpu.VMEM((_CK, h), jnp.float32),
          pltpu.VMEM_SHARED((acc_rows, h), jnp.float32),
      ])(dst3, zac, ones)[0]


def _prep_call(x, y, nb, bn):
  n, d = x.shape
  h = d // 2

  def body(x_ref, y_ref, o_ref):
    p = x_ref[...] * y_ref[...]
    o_ref[0] = p[:, :h]
    o_ref[1] = p[:, h:]

  return pl.pallas_call(
      body,
      grid=(nb,),
      in_specs=[pl.BlockSpec((bn, d), lambda i: (i, 0)),
                pl.BlockSpec((bn, d), lambda i: (i, 0))],
      out_specs=pl.BlockSpec((2, bn, h), lambda i: (0, i, 0)),
      out_shape=jax.ShapeDtypeStruct((2, n, h), jnp.float32),
  )(x, y)


def _mm_call(tabs, deg, w, b, specs, n, nb, bn):
  n_tabs = len(tabs)
  n_ops = len(specs)
  d = w.shape[-1]
  h = d // 2
  acc_rows = tabs[0].shape[1]

  def body(*refs):
    tab_refs = refs[:n_tabs]
    d_ref = refs[n_tabs]
    w_ref = refs[n_tabs + 1]
    b_ref = refs[n_tabs + 2]
    y_refs = refs[n_tabs + 3:n_tabs + 3 + n_ops]
    st_ref = refs[n_tabs + 3 + n_ops]
    acc_ref = refs[-1]
    i = pl.program_id(0)
    rs = 1.0 / jnp.clip(d_ref[0, :, 0:1] + d_ref[1, :, 0:1], 1.0, None)
    parts = []
    for k, (ti, wi) in enumerate(specs):
      a = tab_refs[ti]
      y = (jnp.dot(a[0], w_ref[wi, :h, :], preferred_element_type=jnp.float32)
           + jnp.dot(a[1], w_ref[wi, h:, :],
                     preferred_element_type=jnp.float32))
      y = y * rs + b_ref[wi:wi + 1]
      y_refs[k][...] = y
      parts.append(jnp.sum(y, axis=0, keepdims=True))
      parts.append(jnp.sum(y * y, axis=0, keepdims=True))
    ps = jnp.concatenate(parts, axis=0)

    @pl.when(i == 0)
    def _():
      acc_ref[...] = ps

    @pl.when(i > 0)
    def _():
      acc_ref[...] = acc_ref[...] + ps

    @pl.when(i == nb - 1)
    def _():
      st_ref[...] = acc_ref[...]

  in_specs = (
      [pl.BlockSpec((2, bn, h), lambda i: (0, i, 0)) for _ in range(n_tabs)]
      + [pl.BlockSpec((2, bn, 128), lambda i: (0, i, 0)),
         pl.BlockSpec(w.shape, lambda i: (0, 0, 0)),
         pl.BlockSpec(b.shape, lambda i: (0, 0))])
  out_specs = (
      [pl.BlockSpec((bn, d), lambda i: (i, 0)) for _ in range(n_ops)]
      + [pl.BlockSpec((2 * n_ops, d), lambda i: (0, 0))])
  out_shape = ([jax.ShapeDtypeStruct((n, d), jnp.float32)
                for _ in range(n_ops)]
               + [jax.ShapeDtypeStruct((2 * n_ops, d), jnp.float32)])
  res = pl.pallas_call(
      body,
      grid=(nb,),
      in_specs=in_specs,
      out_specs=out_specs,
      out_shape=out_shape,
      scratch_shapes=[pltpu.VMEM((2 * n_ops, d), jnp.float32)],
  )(*tabs, deg, w, b)
  return res[:n_ops], res[n_ops]


def _bn(y, st_ref, row, g_ref, be_ref, grow, n):
  mu = st_ref[2 * row:2 * row + 1] / n
  var = st_ref[2 * row + 1:2 * row + 2] / n - mu * mu
  inv = g_ref[grow:grow + 1] * lax.rsqrt(var + _EPS)
  return jnp.maximum((y - mu) * inv + be_ref[grow:grow + 1], 0.0)


def _stage1_call(y0, st0, g, be, x, n, nb, bn):
  d = y0.shape[1]
  h = d // 2

  def body(y_ref, st_ref, g_ref, be_ref, x_ref, z_ref, p1_ref, p2_ref):
    z = _bn(y_ref[...], st_ref, 0, g_ref, be_ref, 0, n)
    z_ref[...] = z
    p1 = x_ref[...] * z
    p2 = z * z
    p1_ref[0] = p1[:, :h]
    p1_ref[1] = p1[:, h:]
    p2_ref[0] = p2[:, :h]
    p2_ref[1] = p2[:, h:]

  return pl.pallas_call(
      body,
      grid=(nb,),
      in_specs=[pl.BlockSpec((bn, d), lambda i: (i, 0)),
                pl.BlockSpec(st0.shape, lambda i: (0, 0)),
                pl.BlockSpec(g.shape, lambda i: (0, 0)),
                pl.BlockSpec(be.shape, lambda i: (0, 0)),
                pl.BlockSpec((bn, d), lambda i: (i, 0))],
      out_specs=[pl.BlockSpec((bn, d), lambda i: (i, 0)),
                 pl.BlockSpec((2, bn, h), lambda i: (0, i, 0)),
                 pl.BlockSpec((2, bn, h), lambda i: (0, i, 0))],
      out_shape=[jax.ShapeDtypeStruct((n, d), jnp.float32),
                 jax.ShapeDtypeStruct((2, n, h), jnp.float32),
                 jax.ShapeDtypeStruct((2, n, h), jnp.float32)],
  )(y0, st0, g, be, x)


def _stage2_call(y10, y11, y20, st, g, be, z, n, nb, bn):
  d = y10.shape[1]
  h = d // 2

  def body(y10_r, y11_r, y20_r, st_ref, g_ref, be_ref, z_ref,
           s2_ref, h20_ref, p3_ref):
    h10 = _bn(y10_r[...], st_ref, 0, g_ref, be_ref, 1, n)
    h11 = _bn(y11_r[...], st_ref, 1, g_ref, be_ref, 2, n)
    h20 = _bn(y20_r[...], st_ref, 2, g_ref, be_ref, 3, n)
    s2 = h10 + h11
    s2_ref[...] = s2
    h20_ref[...] = h20
    p3 = s2 * z_ref[...]
    p3_ref[0] = p3[:, :h]
    p3_ref[1] = p3[:, h:]

  return pl.pallas_call(
      body,
      grid=(nb,),
      in_specs=[pl.BlockSpec((bn, d), lambda i: (i, 0)) for _ in range(3)]
      + [pl.BlockSpec(st.shape, lambda i: (0, 0)),
         pl.BlockSpec(g.shape, lambda i: (0, 0)),
         pl.BlockSpec(be.shape, lambda i: (0, 0)),
         pl.BlockSpec((bn, d), lambda i: (i, 0))],
      out_specs=[pl.BlockSpec((bn, d), lambda i: (i, 0)),
                 pl.BlockSpec((bn, d), lambda i: (i, 0)),
                 pl.BlockSpec((2, bn, h), lambda i: (0, i, 0))],
      out_shape=[jax.ShapeDtypeStruct((n, d), jnp.float32),
                 jax.ShapeDtypeStruct((n, d), jnp.float32),
                 jax.ShapeDtypeStruct((2, n, h), jnp.float32)],
  )(y10, y11, y20, st, g, be, z)


def _stage3_call(y22, st22, g, be, h20, z, s2, wc, bc, n, nb, bn):
  d = y22.shape[1]

  def body(y22_r, st_ref, g_ref, be_ref, h20_r, z_r, s2_r, wc_ref, bc_ref,
           yc_ref, stc_ref, acc_ref):
    i = pl.program_id(0)
    h22 = _bn(y22_r[...], st_ref, 0, g_ref, be_ref, 4, n)
    s3 = h20_r[...] + h22
    yc = (jnp.dot(z_r[...], wc_ref[:d, :], preferred_element_type=jnp.float32)
          + jnp.dot(s2_r[...], wc_ref[d:2 * d, :],
                    preferred_element_type=jnp.float32)
          + jnp.dot(s3, wc_ref[2 * d:, :], preferred_element_type=jnp.float32)
          + bc_ref[0:1])
    yc_ref[...] = yc
    ps = jnp.concatenate([jnp.sum(yc, axis=0, keepdims=True),
                          jnp.sum(yc * yc, axis=0, keepdims=True)], axis=0)

    @pl.when(i == 0)
    def _():
      acc_ref[...] = ps

    @pl.when(i > 0)
    def _():
      acc_ref[...] = acc_ref[...] + ps

    @pl.when(i == nb - 1)
    def _():
      stc_ref[...] = acc_ref[...]

  return pl.pallas_call(
      body,
      grid=(nb,),
      in_specs=[pl.BlockSpec((bn, d), lambda i: (i, 0)),
                pl.BlockSpec(st22.shape, lambda i: (0, 0)),
                pl.BlockSpec(g.shape, lambda i: (0, 0)),
                pl.BlockSpec(be.shape, lambda i: (0, 0)),
                pl.BlockSpec((bn, d), lambda i: (i, 0)),
                pl.BlockSpec((bn, d), lambda i: (i, 0)),
                pl.BlockSpec((bn, d), lambda i: (i, 0)),
                pl.BlockSpec(wc.shape, lambda i: (0, 0)),
                pl.BlockSpec(bc.shape, lambda i: (0, 0))],
      out_specs=[pl.BlockSpec((bn, d), lambda i: (i, 0)),
                 pl.BlockSpec((2, d), lambda i: (0, 0))],
      out_shape=[jax.ShapeDtypeStruct((n, d), jnp.float32),
                 jax.ShapeDtypeStruct((2, d), jnp.float32)],
      scratch_shapes=[pltpu.VMEM((2, d), jnp.float32)],
  )(y22, st22, g, be, h20, z, s2, wc, bc)


def _final_call(yc, stc, gc, bc, n, nb, bn):
  d = yc.shape[1]

  def body(y_ref, st_ref, g_ref, be_ref, o_ref):
    o_ref[...] = _bn(y_ref[...], st_ref, 0, g_ref, be_ref, 0, n)

  return pl.pallas_call(
      body,
      grid=(nb,),
      in_specs=[pl.BlockSpec((bn, d), lambda i: (i, 0)),
                pl.BlockSpec(stc.shape, lambda i: (0, 0)),
                pl.BlockSpec(gc.shape, lambda i: (0, 0)),
                pl.BlockSpec(bc.shape, lambda i: (0, 0))],
      out_specs=pl.BlockSpec((bn, d), lambda i: (i, 0)),
      out_shape=jax.ShapeDtypeStruct((n, d), jnp.float32),
  )(yc, stc, gc, bc)


def kernel(src_emb, hr, edge_index, W_ops, b_ops, gamma_ops, beta_ops,
           W_cat, b_cat, gamma_c, beta_c):
  n, d = src_emb.shape
  e = edge_index.shape[1]
  h = d // 2

  bn = 1000
  nb = n // bn

  per_tile = -(-e // (_NT * _CK * 16)) * _CK * 16
  n_chunks = per_tile // _CK
  ep = per_tile * _NT
  pad = ep - e
  src = edge_index[0]
  dst = edge_index[1]
  srcp = jnp.concatenate([src, jnp.zeros((pad,), jnp.int32)])
  dstp = jnp.concatenate([dst, jnp.full((pad,), n, jnp.int32)])
  src2 = jnp.stack([srcp, srcp + n]).reshape(_NC, _NT, n_chunks, _CK)
  dst3 = dstp.reshape(_NT, n_chunks, _CK)
  rt = -(-(n + 1) // (_NT * 8)) * 8
  acc_rows = rt * _NT

  b2 = b_ops.reshape(5, d)
  g2 = gamma_ops.reshape(5, d)
  be2 = beta_ops.reshape(5, d)
  bc2 = b_cat.reshape(1, d)
  gc2 = gamma_c.reshape(1, d)
  bec2 = beta_c.reshape(1, d)

  p0 = _prep_call(src_emb, hr, nb, bn)
  deg = _sc_degree(dst3, n_chunks, acc_rows)
  agg0 = _sc_segsum(p0.reshape(2 * n, h), src2, dst3, n_chunks, acc_rows)

  (y0,), st0 = _mm_call([agg0], deg, W_ops, b2, [(0, 0)], n, nb, bn)
  z, p1, p2 = _stage1_call(y0, st0, g2, be2, src_emb, n, nb, bn)

  agg1 = _sc_segsum(p1.reshape(2 * n, h), src2, dst3, n_chunks, acc_rows)
  agg2 = _sc_segsum(p2.reshape(2 * n, h), src2, dst3, n_chunks, acc_rows)

  (y10, y11, y20), st3 = _mm_call([agg1, agg2], deg, W_ops, b2,
                                  [(0, 1), (1, 2), (0, 3)], n, nb, bn)
  s2, h20, p3 = _stage2_call(y10, y11, y20, st3, g2, be2, z, n, nb, bn)

  agg3 = _sc_segsum(p3.reshape(2 * n, h), src2, dst3, n_chunks, acc_rows)

  (y22,), st22 = _mm_call([agg3], deg, W_ops, b2, [(0, 4)], n, nb, bn)
  yc, stc = _stage3_call(y22, st22, g2, be2, h20, z, s2, W_cat, bc2,
                         n, nb, bn)
  return _final_call(yc, stc, gc2, bec2, n, nb, bn)
